# Initial kernel scaffold; baseline (speedup 1.0000x reference)
import jax
import jax.numpy as jnp
from jax.experimental import pallas as pl


def kernel(src, index):
    raise NotImplementedError("write your pallas kernel here")


# speedup vs baseline: 5.9311x; 5.9311x over previous
import functools

import jax
import jax.numpy as jnp
from jax import lax
from jax.experimental import pallas as pl
from jax.experimental.pallas import tpu as pltpu
from jax.experimental.pallas import tpu_sc as plsc

_NUM_CORES = 2
_NUM_SUBCORES = 16
_NUM_WORKERS = _NUM_CORES * _NUM_SUBCORES
_CHUNK = 1000


@functools.lru_cache(maxsize=None)
def _make_gather(V, D, B):
  del V
  assert B % _NUM_WORKERS == 0
  b_per_w = B // _NUM_WORKERS
  assert b_per_w % _CHUNK == 0
  n_chunks = b_per_w // _CHUNK
  mesh = plsc.VectorSubcoreMesh(core_axis_name="c", subcore_axis_name="s")

  @functools.partial(
      pl.kernel,
      mesh=mesh,
      out_type=jax.ShapeDtypeStruct((B, D), jnp.float32),
      scratch_types=[
          pltpu.VMEM((_CHUNK,), jnp.int32),
          pltpu.VMEM((_CHUNK, D), jnp.float32),
          pltpu.SemaphoreType.DMA,
      ],
  )
  def gather_kernel(table_hbm, idx_hbm, out_hbm, idx_v, rows_v, sem):
    wid = lax.axis_index("s") * _NUM_CORES + lax.axis_index("c")
    base = wid * b_per_w
    for i in range(n_chunks):
      off = base + i * _CHUNK
      pltpu.sync_copy(idx_hbm.at[pl.ds(off, _CHUNK)], idx_v)
      pltpu.async_copy(table_hbm.at[idx_v], rows_v, sem).wait()
      pltpu.sync_copy(rows_v, out_hbm.at[pl.ds(off, _CHUNK)])

  return gather_kernel


def kernel(src, index):
  idx = index.astype(jnp.int32)
  return _make_gather(src.shape[0], src.shape[1], idx.shape[0])(src, idx)


# speedup vs baseline: 5.9774x; 1.0078x over previous
import functools

import jax
import jax.numpy as jnp
from jax import lax
from jax.experimental import pallas as pl
from jax.experimental.pallas import tpu as pltpu
from jax.experimental.pallas import tpu_sc as plsc

_NUM_CORES = 2
_NUM_SUBCORES = 16
_NUM_WORKERS = _NUM_CORES * _NUM_SUBCORES
_CHUNK = 400


@functools.lru_cache(maxsize=None)
def _make_gather(V, D, B):
  del V
  assert B % _NUM_WORKERS == 0
  b_per_w = B // _NUM_WORKERS
  assert b_per_w % _CHUNK == 0
  n_chunks = b_per_w // _CHUNK
  mesh = plsc.VectorSubcoreMesh(core_axis_name="c", subcore_axis_name="s")

  @functools.partial(
      pl.kernel,
      mesh=mesh,
      out_type=jax.ShapeDtypeStruct((B, D), jnp.float32),
      scratch_types=[
          pltpu.VMEM((_CHUNK,), jnp.int32),
          pltpu.VMEM((_CHUNK,), jnp.int32),
          pltpu.VMEM((_CHUNK, D), jnp.float32),
          pltpu.VMEM((_CHUNK, D), jnp.float32),
          pltpu.SemaphoreType.DMA,
          pltpu.SemaphoreType.DMA,
          pltpu.SemaphoreType.DMA,
          pltpu.SemaphoreType.DMA,
      ],
  )
  def gather_kernel(table_hbm, idx_hbm, out_hbm, idx_v0, idx_v1, rows_v0,
                    rows_v1, sg0, sg1, ss0, ss1):
    wid = lax.axis_index("s") * _NUM_CORES + lax.axis_index("c")
    base = wid * b_per_w
    idx_v = (idx_v0, idx_v1)
    rows_v = (rows_v0, rows_v1)
    sg = (sg0, sg1)
    ss = (ss0, ss1)

    def start_gather(i):
      b = i % 2
      pltpu.sync_copy(idx_hbm.at[pl.ds(base + i * _CHUNK, _CHUNK)], idx_v[b])
      return pltpu.async_copy(table_hbm.at[idx_v[b]], rows_v[b], sg[b])

    gathers = [None] * n_chunks
    stores = [None] * n_chunks
    gathers[0] = start_gather(0)
    for i in range(n_chunks):
      b = i % 2
      if i + 1 < n_chunks:
        if i >= 1:
          stores[i - 1].wait()
        gathers[i + 1] = start_gather(i + 1)
      gathers[i].wait()
      stores[i] = pltpu.async_copy(
          rows_v[b], out_hbm.at[pl.ds(base + i * _CHUNK, _CHUNK)], ss[b])
    stores[n_chunks - 1].wait()

  return gather_kernel


def kernel(src, index):
  idx = index.astype(jnp.int32)
  return _make_gather(src.shape[0], src.shape[1], idx.shape[0])(src, idx)
